# Initial kernel scaffold; baseline (speedup 1.0000x reference)
import jax
import jax.numpy as jnp
from jax.experimental import pallas as pl


def kernel(indices, linear_in, emb_table, W, b):
    raise NotImplementedError("write your pallas kernel here")


# speedup vs baseline: 4.0411x; 4.0411x over previous
import functools

import jax
import jax.numpy as jnp
from jax import lax
from jax.experimental import pallas as pl
from jax.experimental.pallas import tpu as pltpu
from jax.experimental.pallas import tpu_sc as plsc

_NUM_CORES = 2
_NUM_SUBCORES = 16
_LANES = 16
_NW = _NUM_CORES * _NUM_SUBCORES

_B, _S = 16384, 200
_V, _D = 50, 12
_N = _B * _S
_PER_W = _N // _NW
_CHUNK = 2048
_STEPS = _CHUNK // _LANES
_NCHUNK = _PER_W // _CHUNK


def _gather_body(idx_hbm, tab_hbm, out_hbm, tab_v, idx_v, out_v):
    cid = lax.axis_index("c")
    sid = lax.axis_index("s")
    wid = sid * _NUM_CORES + cid
    base = wid * _PER_W

    pltpu.sync_copy(tab_hbm, tab_v)

    lane = lax.iota(jnp.int32, _LANES)

    def chunk_body(c, _):
        cbase = base + c * _CHUNK
        pltpu.sync_copy(idx_hbm.at[pl.ds(cbase, _CHUNK)], idx_v)

        def step(s, _):
            ind16 = idx_v[pl.ds(s * _LANES, _LANES)]
            src = ind16 * _D
            dst = lane * _D + s * (_LANES * _D)
            for d in range(_D):
                vals = plsc.load_gather(tab_v, [src + d])
                plsc.store_scatter(out_v, [dst + d], vals)
            return 0

        lax.fori_loop(0, _STEPS, step, 0, unroll=False)
        pltpu.sync_copy(out_v, out_hbm.at[pl.ds(cbase * _D, _CHUNK * _D)])
        return 0

    lax.fori_loop(0, _NCHUNK, chunk_body, 0, unroll=False)


@jax.jit
def _sc_gather(idx_flat, tab_flat):
    mesh = plsc.VectorSubcoreMesh(
        core_axis_name="c", subcore_axis_name="s",
        num_cores=_NUM_CORES, num_subcores=_NUM_SUBCORES,
    )
    return pl.kernel(
        _gather_body,
        out_type=jax.ShapeDtypeStruct((_N * _D,), jnp.float32),
        mesh=mesh,
        compiler_params=pltpu.CompilerParams(needs_layout_passes=False),
        scratch_types=[
            pltpu.VMEM((_V * _D,), jnp.float32),
            pltpu.VMEM((_CHUNK,), jnp.int32),
            pltpu.VMEM((_CHUNK * _D,), jnp.float32),
        ],
    )(idx_flat, tab_flat)


def _linear_body(x_ref, wt_ref, b_ref, o_ref):
    o_ref[...] = (
        jnp.dot(x_ref[...], wt_ref[...], preferred_element_type=jnp.float32)
        + b_ref[...]
    )


@jax.jit
def _tc_linear(x, wt, b2):
    return pl.pallas_call(
        _linear_body,
        out_shape=jax.ShapeDtypeStruct(x.shape, jnp.float32),
    )(x, wt, b2)


def kernel(indices, linear_in, emb_table, W, b):
    idx_flat = indices.reshape(_N)
    tab_flat = emb_table.reshape(_V * _D)
    a = _sc_gather(idx_flat, tab_flat).reshape(_B, _S, _D)
    q = _tc_linear(linear_in, W.T, b.reshape(1, -1))
    return (a, q)
